# baseline (speedup 1.0000x reference)
import functools

import jax
import jax.numpy as jnp
from jax import lax
from jax.experimental import pallas as pl
from jax.experimental.pallas import tpu as pltpu
from jax.experimental.pallas import tpu_sc as plsc

_TAU = 5.0
_LAMBDA_MSE = 0.5

_S = 16384
_K = 200
_NW = 32
_CHUNK = 6400


def _sc_gather(idx_flat, theta):
    n = idx_flat.shape[0]
    per_w = n // _NW
    chunk = _CHUNK
    while per_w % chunk or (per_w // chunk) % 2:
        chunk //= 2
    steps = per_w // chunk
    mesh = plsc.VectorSubcoreMesh(core_axis_name="c", subcore_axis_name="s")

    @functools.partial(
        pl.kernel,
        mesh=mesh,
        out_type=jax.ShapeDtypeStruct((n,), jnp.float32),
        scratch_types=[
            pltpu.VMEM((chunk,), jnp.int32),
            pltpu.VMEM((chunk,), jnp.int32),
            pltpu.VMEM((chunk,), jnp.float32),
            pltpu.VMEM((chunk,), jnp.float32),
            pltpu.SemaphoreType.DMA,
            pltpu.SemaphoreType.DMA,
            pltpu.SemaphoreType.DMA,
            pltpu.SemaphoreType.DMA,
            pltpu.SemaphoreType.DMA,
            pltpu.SemaphoreType.DMA,
        ],
    )
    def gather_kernel(idx_hbm, theta_hbm, out_hbm, idx0, idx1, val0, val1,
                      si0, si1, sg0, sg1, sw0, sw1):
        nc = lax.axis_size("c")
        wid = lax.axis_index("s") * nc + lax.axis_index("c")
        base = wid * per_w
        idx_v = (idx0, idx1)
        val_v = (val0, val1)
        semi = (si0, si1)
        semg = (sg0, sg1)
        semw = (sw0, sw1)

        def idx_start(g, p):
            pltpu.async_copy(
                idx_hbm.at[pl.ds(base + g * chunk, chunk)], idx_v[p], semi[p]
            )

        idx_start(0, 0)

        def phase(g, p):
            q = 1 - p
            pltpu.make_async_copy(
                idx_hbm.at[pl.ds(0, chunk)], idx_v[p], semi[p]
            ).wait()
            @pl.when(g >= 2)
            def _():
                pltpu.make_async_copy(
                    val_v[p], out_hbm.at[pl.ds(0, chunk)], semw[p]
                ).wait()
            pltpu.async_copy(theta_hbm.at[idx_v[p]], val_v[p], semg[p])
            @pl.when(g >= 1)
            def _():
                pltpu.make_async_copy(
                    theta_hbm.at[idx_v[q]], val_v[q], semg[q]
                ).wait()
                pltpu.async_copy(
                    val_v[q],
                    out_hbm.at[pl.ds(base + (g - 1) * chunk, chunk)],
                    semw[q],
                )
            @pl.when(g + 1 < steps)
            def _():
                idx_start(g + 1, q)

        def pair(g2, carry):
            phase(2 * g2, 0)
            phase(2 * g2 + 1, 1)
            return carry

        lax.fori_loop(0, steps // 2, pair, 0)
        pltpu.make_async_copy(theta_hbm.at[idx1], val1, sg1).wait()
        pltpu.async_copy(
            val1, out_hbm.at[pl.ds(base + (steps - 1) * chunk, chunk)], sw1
        )
        pltpu.make_async_copy(val0, out_hbm.at[pl.ds(0, chunk)], sw0).wait()
        pltpu.make_async_copy(val1, out_hbm.at[pl.ds(0, chunk)], sw1).wait()

    return gather_kernel(idx_flat, theta)


def _make_tc_body(final):
    def tc_body(a_ref, prev_ref, lens_ref, b_ref, t_ref, sc_ref, out_ref,
                acc_ref):
        i = pl.program_id(0)
        n = pl.num_programs(0)

        @pl.when(i == 0)
        def _init():
            acc_ref[0] = prev_ref[0]
            acc_ref[1] = prev_ref[1]
            acc_ref[2] = prev_ref[2]
            acc_ref[3] = prev_ref[3]

        t = t_ref[...] * _TAU
        bs = t.shape[0]
        kio = lax.broadcasted_iota(jnp.int32, (bs, _K), 1)
        mask = kio < lens_ref[...]
        maskf = mask.astype(jnp.float32)

        e = jnp.where(mask, jnp.exp(t), 0.0)
        rj = lax.broadcasted_iota(jnp.int32, (_K, _K), 0)
        ci = lax.broadcasted_iota(jnp.int32, (_K, _K), 1)
        tri = (rj >= ci).astype(jnp.float32)
        cumexp = jnp.dot(e, tri, preferred_element_type=jnp.float32)
        logc = jnp.log(cumexp + 1e-12)

        sum_t = jnp.sum(t * maskf)
        sum_lc = jnp.sum(logc * maskf)

        sc = sc_ref[...]
        w = jnp.maximum(1.0 / (1.0 + jnp.exp(-(sc - 0.5))), 0.1)
        wm = w * maskf
        pred = a_ref[0, 0] * t + b_ref[...]
        d = pred - sc * _TAU
        sum_se = jnp.sum(d * d * wm)
        sum_wm = jnp.sum(wm)

        acc_ref[0] += sum_t
        acc_ref[1] += sum_lc
        acc_ref[2] += sum_se
        acc_ref[3] += sum_wm

        if final:
            @pl.when(i == n - 1)
            def _fin():
                nll = -(acc_ref[0] - acc_ref[1]) / float(_S)
                mse = acc_ref[2] / acc_ref[3]
                out_ref[0] = (1.0 - _LAMBDA_MSE) * nll + _LAMBDA_MSE * mse
                out_ref[1] = nll
                out_ref[2] = mse
        else:
            @pl.when(i == n - 1)
            def _fin():
                out_ref[0] = acc_ref[0]
                out_ref[1] = acc_ref[1]
                out_ref[2] = acc_ref[2]
                out_ref[3] = acc_ref[3]

    return tc_body


def _tc_reduce(t, scores, lens2d, a2d, b2d, prev, block0, final):
    ns = t.shape[0]
    bs = 512
    grid = ns // bs
    out = pl.pallas_call(
        _make_tc_body(final),
        grid=(grid,),
        in_specs=[
            pl.BlockSpec((1, 1), lambda i: (0, 0), memory_space=pltpu.SMEM),
            pl.BlockSpec(memory_space=pltpu.SMEM),
            pl.BlockSpec((bs, 1), lambda i: (i + block0, 0)),
            pl.BlockSpec((bs, 1), lambda i: (i + block0, 0)),
            pl.BlockSpec((bs, _K), lambda i: (i, 0)),
            pl.BlockSpec((bs, _K), lambda i: (i + block0, 0)),
        ],
        out_specs=pl.BlockSpec(memory_space=pltpu.SMEM),
        out_shape=jax.ShapeDtypeStruct((3 if final else 4,), jnp.float32),
        scratch_shapes=[pltpu.SMEM((4,), jnp.float32)],
    )(
        a2d,
        prev,
        lens2d,
        b2d,
        t,
        scores,
    )
    return out


_SPLITS = (1024, 5120, 5120, 4096, 1024)


def kernel(slates, scores, lens, theta, a, b_s):
    a2d = jnp.asarray(a, jnp.float32).reshape(1, 1)
    lens2d = lens.reshape(_S, 1)
    b2d = b_s.reshape(_S, 1)
    offs = [0]
    for ns in _SPLITS:
        offs.append(offs[-1] + ns)
    gs = [
        _sc_gather(slates[offs[i]:offs[i + 1]].reshape(-1), theta)
        for i in range(len(_SPLITS))
    ]
    acc = jnp.zeros((4,), jnp.float32)
    for i, ns in enumerate(_SPLITS):
        acc = _tc_reduce(
            gs[i].reshape(ns, _K), scores, lens2d, a2d, b2d, acc,
            offs[i] // 512, i == len(_SPLITS) - 1
        )
    return (acc[0], acc[1], acc[2])
